# baseline (speedup 1.0000x reference)
import jax
import jax.numpy as jnp
from jax.experimental import pallas as pl
from jax.experimental.pallas import tpu as pltpu

N = 4096
BA = 256
BB = 1024
NA = N // BA
NB = N // BB
P_B = NA
P_C = NA + NB


def _gcn_kernel(adj_ref, x_ref, w0_ref, w1_ref, b0_ref, b1_ref, s_in_ref,
                wch_ref, wcs_ref, bc_ref, h_ref, y_ref,
                adjb_scr, s0_scr, s1_scr, d_scr):
    i = pl.program_id(0)

    @pl.when(i < P_B)
    def _phase_a():
        a = adj_ref[...]
        adjb_scr[pl.ds(i * BA, BA), :] = a.astype(jnp.bfloat16)
        d = jax.lax.rsqrt(1.0 + jnp.sum(a, axis=1, keepdims=True))
        d_scr[pl.ds(i * BA, BA), :] = d
        s0 = d * jnp.dot(x_ref[...], w0_ref[...],
                         preferred_element_type=jnp.float32)
        s0_scr[pl.ds(i * BA, BA), :] = s0.astype(jnp.bfloat16)

    @pl.when(jnp.logical_and(i >= P_B, i < P_C))
    def _phase_b():
        r = (i - P_B) * BB
        t = jnp.dot(adjb_scr[pl.ds(r, BB), :], s0_scr[...],
                    preferred_element_type=jnp.float32)
        own = s0_scr[pl.ds(r, BB), :].astype(jnp.float32)
        h0 = d_scr[pl.ds(r, BB), :] * (t + own) + b0_ref[...]
        h0 = jnp.where(h0 >= 0, h0, 0.01 * h0)
        s1 = d_scr[pl.ds(r, BB), :] * jnp.dot(
            h0.astype(jnp.bfloat16), w1_ref[...],
            preferred_element_type=jnp.float32)
        s1_scr[pl.ds(r, BB), :] = s1.astype(jnp.bfloat16)

    @pl.when(i >= P_C)
    def _phase_c():
        r = (i - P_C) * BB
        t = jnp.dot(adjb_scr[pl.ds(r, BB), :], s1_scr[...],
                    preferred_element_type=jnp.float32)
        own = s1_scr[pl.ds(r, BB), :].astype(jnp.float32)
        h = d_scr[pl.ds(r, BB), :] * (t + own) + b1_ref[...]
        h_ref[...] = h
        logits = (jnp.dot(h, wch_ref[...], preferred_element_type=jnp.float32)
                  + jnp.dot(s_in_ref[...], wcs_ref[...],
                            preferred_element_type=jnp.float32)
                  + bc_ref[...])
        m = jnp.max(logits, axis=1, keepdims=True)
        e = jnp.exp(logits - m)
        y_ref[...] = e / jnp.sum(e, axis=1, keepdims=True)


def kernel(x, adj, S, W0, b0, W1, b1, Wc, bc):
    in_dim = x.shape[1]
    hid = W0.shape[1]
    f_dim = W1.shape[1]
    s_dim = S.shape[1]
    c_dim = Wc.shape[1]

    def a_map(i):
        return (jnp.minimum(i, NA - 1), 0)

    def c_map(i):
        return (jnp.clip(i - P_C, 0, NB - 1), 0)

    h, y = pl.pallas_call(
        _gcn_kernel,
        grid=(NA + NB + NB,),
        in_specs=[
            pl.BlockSpec((BA, N), a_map),
            pl.BlockSpec((BA, in_dim), a_map),
            pl.BlockSpec((in_dim, hid), lambda i: (0, 0)),
            pl.BlockSpec((hid, f_dim), lambda i: (0, 0)),
            pl.BlockSpec((1, hid), lambda i: (0, 0)),
            pl.BlockSpec((1, f_dim), lambda i: (0, 0)),
            pl.BlockSpec((BB, s_dim), c_map),
            pl.BlockSpec((f_dim, c_dim), lambda i: (0, 0)),
            pl.BlockSpec((s_dim, c_dim), lambda i: (0, 0)),
            pl.BlockSpec((1, c_dim), lambda i: (0, 0)),
        ],
        out_specs=[
            pl.BlockSpec((BB, f_dim), c_map),
            pl.BlockSpec((BB, c_dim), c_map),
        ],
        out_shape=[
            jax.ShapeDtypeStruct((N, f_dim), jnp.float32),
            jax.ShapeDtypeStruct((N, c_dim), jnp.float32),
        ],
        scratch_shapes=[
            pltpu.VMEM((N, N), jnp.bfloat16),
            pltpu.VMEM((N, hid), jnp.bfloat16),
            pltpu.VMEM((N, f_dim), jnp.bfloat16),
            pltpu.VMEM((N, 1), jnp.float32),
        ],
        compiler_params=pltpu.CompilerParams(
            dimension_semantics=("arbitrary",)),
    )(adj, x, W0, W1.astype(jnp.bfloat16), b0.reshape(1, hid), b1.reshape(1, f_dim), S,
      Wc[:f_dim], Wc[f_dim:], bc.reshape(1, c_dim))

    return (h, y)
